# Initial kernel scaffold; baseline (speedup 1.0000x reference)
import jax
import jax.numpy as jnp
from jax.experimental import pallas as pl


def kernel(x, edge_index, edge_attr, We1, be1, W1a, b1a, g1, beta1, W1b, b1b, We2, be2, W2a, b2a, g2, beta2, W2b, b2b):
    raise NotImplementedError("write your pallas kernel here")


# speedup vs baseline: 2.9185x; 2.9185x over previous
import functools

import jax
import jax.numpy as jnp
from jax import lax
from jax.experimental import pallas as pl
from jax.experimental.pallas import tpu as pltpu
from jax.experimental.pallas import tpu_sc as plsc

N = 10000
E = 320000
D = 128
ED = 16

NC = 2
NS = 16
NW = NC * NS
EPT = E // NW
C = 80
NCHUNK = EPT // C
ZROWS = 80
ZCH = N // ZROWS
LANES = 16


def _sc_edge_kernel(x_hbm, src_hbm, dst_hbm, e_hbm, out_hbm,
                    sidx, didx, xr, er, zbuf, acc, sem):
    cid = lax.axis_index("c")
    sid = lax.axis_index("s")
    wid = cid * NS + sid

    zero = jnp.zeros((LANES,), jnp.float32)

    def zfill(i, _):
        r = i // (D // LANES)
        col = (i % (D // LANES)) * LANES
        zbuf[r, pl.ds(col, LANES)] = zero
        return 0

    lax.fori_loop(0, ZROWS * (D // LANES), zfill, 0)

    def zcopy(t, _):
        j = sid + t * NS

        @pl.when(j < ZCH)
        def _():
            pltpu.sync_copy(zbuf, acc.at[pl.ds(j * ZROWS, ZROWS)])

        return 0

    lax.fori_loop(0, (ZCH + NS - 1) // NS, zcopy, 0)
    plsc.subcore_barrier()

    base = wid * EPT

    def chunk(k, _):
        b = base + k * C
        pltpu.sync_copy(src_hbm.at[pl.ds(b, C)], sidx)
        pltpu.sync_copy(dst_hbm.at[pl.ds(b, C)], didx)
        gather = pltpu.async_copy(x_hbm.at[sidx], xr, sem)
        pltpu.sync_copy(e_hbm.at[pl.ds(b, C)], er)
        gather.wait()

        def row(i, _):
            for j in range(D // LANES):
                s = pl.ds(j * LANES, LANES)
                er[i, s] = jnp.maximum(xr[i, s] + er[i, s], 0.0)
            return 0

        lax.fori_loop(0, C, row, 0)
        pltpu.sync_copy(er, acc.at[didx], add=True)
        return 0

    lax.fori_loop(0, NCHUNK, chunk, 0)
    plsc.subcore_barrier()

    def wcopy(t, _):
        j = sid + t * NS

        @pl.when(j < ZCH)
        def _():
            pltpu.sync_copy(acc.at[pl.ds(j * ZROWS, ZROWS)],
                            out_hbm.at[pl.ds(cid * N + j * ZROWS, ZROWS)])

        return 0

    lax.fori_loop(0, (ZCH + NS - 1) // NS, wcopy, 0)


def _sc_edge(x, src, dst, e):
    mesh = plsc.VectorSubcoreMesh(core_axis_name="c", subcore_axis_name="s",
                                  num_cores=NC, num_subcores=NS)
    f = pl.kernel(
        _sc_edge_kernel,
        mesh=mesh,
        out_type=jax.ShapeDtypeStruct((NC * N, D), jnp.float32),
        scratch_types=[
            pltpu.VMEM((C,), jnp.int32),
            pltpu.VMEM((C,), jnp.int32),
            pltpu.VMEM((C, D), jnp.float32),
            pltpu.VMEM((C, D), jnp.float32),
            pltpu.VMEM((ZROWS, D), jnp.float32),
            pltpu.VMEM_SHARED((N, D), jnp.float32),
            pltpu.SemaphoreType.DMA,
        ],
    )
    return f(x, src, dst, e)


def _tc_edge_body(ea_ref, we_ref, be_ref, out_ref):
    out_ref[...] = (
        jnp.dot(ea_ref[...], we_ref[...], preferred_element_type=jnp.float32)
        + be_ref[...]
    )


def _tc_edge(edge_attr, We, be):
    BE = 4000
    return pl.pallas_call(
        _tc_edge_body,
        grid=(E // BE,),
        in_specs=[
            pl.BlockSpec((BE, ED), lambda i: (i, 0)),
            pl.BlockSpec((ED, D), lambda i: (0, 0)),
            pl.BlockSpec((1, D), lambda i: (0, 0)),
        ],
        out_specs=pl.BlockSpec((BE, D), lambda i: (i, 0)),
        out_shape=jax.ShapeDtypeStruct((E, D), jnp.float32),
    )(edge_attr, We, be.reshape(1, D))


def _tc_node_body(x_ref, agg_ref, wa_ref, ba_ref, g_ref, beta_ref,
                  wb_ref, bb_ref, out_ref):
    x = x_ref[...]
    h = x + agg_ref[0] + agg_ref[1]
    h1 = jnp.dot(h, wa_ref[...], preferred_element_type=jnp.float32) + ba_ref[...]
    mean = jnp.mean(h1, axis=0, keepdims=True)
    var = jnp.mean((h1 - mean) ** 2, axis=0, keepdims=True)
    h1 = (h1 - mean) / jnp.sqrt(var + 1e-5) * g_ref[...] + beta_ref[...]
    h1 = jnp.maximum(h1, 0.0)
    out = jnp.dot(h1, wb_ref[...], preferred_element_type=jnp.float32) + bb_ref[...]
    out_ref[...] = jnp.maximum(out + x, 0.0)


def _tc_node(x, agg, Wa, ba, g, beta, Wb, bb):
    return pl.pallas_call(
        _tc_node_body,
        out_shape=jax.ShapeDtypeStruct((N, D), jnp.float32),
    )(x, agg.reshape(NC, N, D), Wa, ba.reshape(1, D), g.reshape(1, D),
      beta.reshape(1, D), Wb, bb.reshape(1, D))


def kernel(x, edge_index, edge_attr, We1, be1, W1a, b1a, g1, beta1, W1b, b1b,
           We2, be2, W2a, b2a, g2, beta2, W2b, b2b):
    src = edge_index[0]
    dst = edge_index[1]
    e1 = _tc_edge(edge_attr, We1, be1)
    e2 = _tc_edge(edge_attr, We2, be2)
    agg1 = _sc_edge(x, src, dst, e1)
    h = _tc_node(x, agg1, W1a, b1a, g1, beta1, W1b, b1b)
    agg2 = _sc_edge(h, src, dst, e2)
    return _tc_node(h, agg2, W2a, b2a, g2, beta2, W2b, b2b)


# speedup vs baseline: 4.5178x; 1.5480x over previous
import functools

import jax
import jax.numpy as jnp
from jax import lax
from jax.experimental import pallas as pl
from jax.experimental.pallas import tpu as pltpu
from jax.experimental.pallas import tpu_sc as plsc

N = 10000
E = 320000
D = 128
ED = 16

NC = 2
NS = 16
NW = NC * NS
EPT = E // NW
C = 80
NCHUNK = EPT // C
ZROWS = 80
ZCH = N // ZROWS
LANES = 16


def _sc_edge_kernel(x_hbm, src_hbm, dst_hbm, e_hbm, out_hbm,
                    sidx0, sidx1, didx0, didx1, xr0, xr1, er0, er1, acc,
                    gsem0, gsem1, esem0, esem1, lsem0, lsem1):
    cid = lax.axis_index("c")
    sid = lax.axis_index("s")
    wid = cid * NS + sid

    sidx = (sidx0, sidx1)
    didx = (didx0, didx1)
    xr = (xr0, xr1)
    er = (er0, er1)
    gsem = (gsem0, gsem1)
    esem = (esem0, esem1)
    lsem = (lsem0, lsem1)

    zero = jnp.zeros((LANES,), jnp.float32)
    zbuf = er0

    def zfill(i, _):
        r = i // (D // LANES)
        col = (i % (D // LANES)) * LANES
        zbuf[r, pl.ds(col, LANES)] = zero
        return 0

    lax.fori_loop(0, ZROWS * (D // LANES), zfill, 0)

    def zcopy(t, _):
        j = sid + t * NS

        @pl.when(j < ZCH)
        def _():
            pltpu.sync_copy(zbuf, acc.at[pl.ds(j * ZROWS, ZROWS)])

        return 0

    lax.fori_loop(0, (ZCH + NS - 1) // NS, zcopy, 0)
    plsc.subcore_barrier()

    base = wid * EPT

    def issue_idx(k, slot):
        pltpu.async_copy(src_hbm.at[pl.ds(base + k * C, C)], sidx[slot],
                         lsem[slot])
        pltpu.async_copy(dst_hbm.at[pl.ds(base + k * C, C)], didx[slot],
                         lsem[slot])

    def wait_idx(k, slot):
        pltpu.make_async_copy(src_hbm.at[pl.ds(base + k * C, C)],
                              sidx[slot], lsem[slot]).wait()
        pltpu.make_async_copy(dst_hbm.at[pl.ds(base + k * C, C)],
                              didx[slot], lsem[slot]).wait()

    def issue_rows(k, slot):
        pltpu.async_copy(x_hbm.at[sidx[slot]], xr[slot], gsem[slot])
        pltpu.async_copy(e_hbm.at[pl.ds(base + k * C, C)], er[slot],
                         esem[slot])

    def wait_rows(k, slot):
        pltpu.make_async_copy(x_hbm.at[sidx[slot]], xr[slot],
                              gsem[slot]).wait()
        pltpu.make_async_copy(e_hbm.at[pl.ds(base + k * C, C)], er[slot],
                              esem[slot]).wait()

    def compute(slot):
        a = xr[slot]
        b = er[slot]

        def row(i, _):
            for j in range(D // LANES):
                s = pl.ds(j * LANES, LANES)
                b[i, s] = jnp.maximum(a[i, s] + b[i, s], 0.0)
            return 0

        lax.fori_loop(0, C, row, 0)

    issue_idx(0, 0)
    wait_idx(0, 0)
    issue_rows(0, 0)
    issue_idx(1, 1)

    def chunk2(k2, _):
        for p in range(2):
            k = k2 * 2 + p
            o = (p + 1) % 2

            @pl.when(k < NCHUNK)
            def _():
                wait_rows(k, p)

                @pl.when(k + 1 < NCHUNK)
                def _():
                    wait_idx(k + 1, o)
                    issue_rows(k + 1, o)

                compute(p)
                pltpu.sync_copy(er[p], acc.at[didx[p]], add=True)

                @pl.when(k + 2 < NCHUNK)
                def _():
                    issue_idx(k + 2, p)

        return 0

    lax.fori_loop(0, (NCHUNK + 1) // 2, chunk2, 0)
    plsc.subcore_barrier()

    def wcopy(t, _):
        j = sid + t * NS

        @pl.when(j < ZCH)
        def _():
            pltpu.sync_copy(acc.at[pl.ds(j * ZROWS, ZROWS)],
                            out_hbm.at[pl.ds(cid * N + j * ZROWS, ZROWS)])

        return 0

    lax.fori_loop(0, (ZCH + NS - 1) // NS, wcopy, 0)


def _sc_edge(x, src, dst, e):
    mesh = plsc.VectorSubcoreMesh(core_axis_name="c", subcore_axis_name="s",
                                  num_cores=NC, num_subcores=NS)
    f = pl.kernel(
        _sc_edge_kernel,
        mesh=mesh,
        out_type=jax.ShapeDtypeStruct((NC * N, D), jnp.float32),
        scratch_types=[
            pltpu.VMEM((C,), jnp.int32),
            pltpu.VMEM((C,), jnp.int32),
            pltpu.VMEM((C,), jnp.int32),
            pltpu.VMEM((C,), jnp.int32),
            pltpu.VMEM((C, D), jnp.float32),
            pltpu.VMEM((C, D), jnp.float32),
            pltpu.VMEM((C, D), jnp.float32),
            pltpu.VMEM((C, D), jnp.float32),
            pltpu.VMEM_SHARED((N, D), jnp.float32),
            pltpu.SemaphoreType.DMA,
            pltpu.SemaphoreType.DMA,
            pltpu.SemaphoreType.DMA,
            pltpu.SemaphoreType.DMA,
            pltpu.SemaphoreType.DMA,
            pltpu.SemaphoreType.DMA,
        ],
    )
    return f(x, src, dst, e)


def _tc_edge_body(ea_ref, we_ref, be_ref, out_ref):
    out_ref[...] = (
        jnp.dot(ea_ref[...], we_ref[...], preferred_element_type=jnp.float32)
        + be_ref[...]
    )


def _tc_edge(edge_attr, We, be):
    BE = 4000
    return pl.pallas_call(
        _tc_edge_body,
        grid=(E // BE,),
        in_specs=[
            pl.BlockSpec((BE, ED), lambda i: (i, 0)),
            pl.BlockSpec((ED, D), lambda i: (0, 0)),
            pl.BlockSpec((1, D), lambda i: (0, 0)),
        ],
        out_specs=pl.BlockSpec((BE, D), lambda i: (i, 0)),
        out_shape=jax.ShapeDtypeStruct((E, D), jnp.float32),
    )(edge_attr, We, be.reshape(1, D))


def _tc_node_body(x_ref, agg_ref, wa_ref, ba_ref, g_ref, beta_ref,
                  wb_ref, bb_ref, out_ref):
    x = x_ref[...]
    h = x + agg_ref[0] + agg_ref[1]
    h1 = jnp.dot(h, wa_ref[...], preferred_element_type=jnp.float32) + ba_ref[...]
    mean = jnp.mean(h1, axis=0, keepdims=True)
    var = jnp.mean((h1 - mean) ** 2, axis=0, keepdims=True)
    h1 = (h1 - mean) / jnp.sqrt(var + 1e-5) * g_ref[...] + beta_ref[...]
    h1 = jnp.maximum(h1, 0.0)
    out = jnp.dot(h1, wb_ref[...], preferred_element_type=jnp.float32) + bb_ref[...]
    out_ref[...] = jnp.maximum(out + x, 0.0)


def _tc_node(x, agg, Wa, ba, g, beta, Wb, bb):
    return pl.pallas_call(
        _tc_node_body,
        out_shape=jax.ShapeDtypeStruct((N, D), jnp.float32),
    )(x, agg.reshape(NC, N, D), Wa, ba.reshape(1, D), g.reshape(1, D),
      beta.reshape(1, D), Wb, bb.reshape(1, D))


def kernel(x, edge_index, edge_attr, We1, be1, W1a, b1a, g1, beta1, W1b, b1b,
           We2, be2, W2a, b2a, g2, beta2, W2b, b2b):
    src = edge_index[0]
    dst = edge_index[1]
    e1 = _tc_edge(edge_attr, We1, be1)
    e2 = _tc_edge(edge_attr, We2, be2)
    agg1 = _sc_edge(x, src, dst, e1)
    h = _tc_node(x, agg1, W1a, b1a, g1, beta1, W1b, b1b)
    agg2 = _sc_edge(h, src, dst, e2)
    return _tc_node(h, agg2, W2a, b2a, g2, beta2, W2b, b2b)


# speedup vs baseline: 4.6604x; 1.0316x over previous
import functools

import jax
import jax.numpy as jnp
from jax import lax
from jax.experimental import pallas as pl
from jax.experimental.pallas import tpu as pltpu
from jax.experimental.pallas import tpu_sc as plsc

N = 10000
E = 320000
D = 128
ED = 16

NC = 2
NS = 16
NW = NC * NS
EPT = E // NW
C = 80
NCHUNK = EPT // C
ZROWS = 80
ZCH = N // ZROWS
LANES = 16


def _sc_edge_kernel(x_hbm, src_hbm, dst_hbm, e_hbm, out_hbm,
                    sidx0, sidx1, didx0, didx1, didx2, xr0, xr1, er0, er1,
                    acc, gsem0, gsem1, esem0, esem1, lsem0, lsem1,
                    dsem0, dsem1, dsem2, ssem0, ssem1):
    cid = lax.axis_index("c")
    sid = lax.axis_index("s")
    wid = cid * NS + sid

    sidx = (sidx0, sidx1)
    didx = (didx0, didx1, didx2)
    xr = (xr0, xr1)
    er = (er0, er1)
    gsem = (gsem0, gsem1)
    esem = (esem0, esem1)
    lsem = (lsem0, lsem1)
    dsem = (dsem0, dsem1, dsem2)
    ssem = (ssem0, ssem1)

    zero = jnp.zeros((LANES,), jnp.float32)
    zbuf = er0

    def zfill(i, _):
        r = i // (D // LANES)
        col = (i % (D // LANES)) * LANES
        zbuf[r, pl.ds(col, LANES)] = zero
        return 0

    lax.fori_loop(0, ZROWS * (D // LANES), zfill, 0)

    def zcopy(t, _):
        j = sid + t * NS

        @pl.when(j < ZCH)
        def _():
            pltpu.sync_copy(zbuf, acc.at[pl.ds(j * ZROWS, ZROWS)])

        return 0

    lax.fori_loop(0, (ZCH + NS - 1) // NS, zcopy, 0)
    plsc.subcore_barrier()

    base = wid * EPT

    def issue_idx(k, s2, s3):
        pltpu.async_copy(src_hbm.at[pl.ds(base + k * C, C)], sidx[s2],
                         lsem[s2])
        pltpu.async_copy(dst_hbm.at[pl.ds(base + k * C, C)], didx[s3],
                         dsem[s3])

    def wait_idx(k, s2, s3):
        pltpu.make_async_copy(src_hbm.at[pl.ds(base + k * C, C)],
                              sidx[s2], lsem[s2]).wait()
        pltpu.make_async_copy(dst_hbm.at[pl.ds(base + k * C, C)],
                              didx[s3], dsem[s3]).wait()

    def issue_rows(k, slot, s2):
        pltpu.async_copy(x_hbm.at[sidx[s2]], xr[slot], gsem[slot])
        pltpu.async_copy(e_hbm.at[pl.ds(base + k * C, C)], er[slot],
                         esem[slot])

    def wait_rows(k, slot, s2):
        pltpu.make_async_copy(x_hbm.at[sidx[s2]], xr[slot],
                              gsem[slot]).wait()
        pltpu.make_async_copy(e_hbm.at[pl.ds(base + k * C, C)], er[slot],
                              esem[slot]).wait()

    def wait_scatter(slot, s3):
        pltpu.make_async_copy(er[slot], acc.at[didx[s3]],
                              ssem[slot]).wait()

    def compute(slot):
        a = xr[slot]
        b = er[slot]

        def row(i2, _):
            for r in range(2):
                for j in range(D // LANES):
                    s = pl.ds(j * LANES, LANES)
                    i = i2 * 2 + r
                    b[i, s] = jnp.maximum(a[i, s] + b[i, s], 0.0)
            return 0

        lax.fori_loop(0, C // 2, row, 0)

    issue_idx(0, 0, 0)
    wait_idx(0, 0, 0)
    issue_rows(0, 0, 0)
    issue_idx(1, 1, 1)

    def chunk6(k6, _):
        for p in range(6):
            k = k6 * 6 + p
            s2 = p % 2
            o2 = (p + 1) % 2
            s3 = p % 3

            @pl.when(k < NCHUNK)
            def _():
                wait_rows(k, s2, s2)

                @pl.when(k + 1 < NCHUNK)
                def _():
                    wait_idx(k + 1, o2, (p + 1) % 3)

                    @pl.when(k >= 1)
                    def _():
                        wait_scatter(o2, (p + 2) % 3)

                    issue_rows(k + 1, o2, o2)

                compute(s2)
                pltpu.async_copy(er[s2], acc.at[didx[s3]], ssem[s2],
                                 add=True)

                @pl.when(k + 2 < NCHUNK)
                def _():
                    issue_idx(k + 2, s2, (p + 2) % 3)

        return 0

    lax.fori_loop(0, (NCHUNK + 5) // 6, chunk6, 0)

    wait_scatter((NCHUNK - 2) % 2, (NCHUNK - 2) % 3)
    wait_scatter((NCHUNK - 1) % 2, (NCHUNK - 1) % 3)
    plsc.subcore_barrier()

    def wcopy(t, _):
        j = sid + t * NS

        @pl.when(j < ZCH)
        def _():
            pltpu.sync_copy(acc.at[pl.ds(j * ZROWS, ZROWS)],
                            out_hbm.at[pl.ds(cid * N + j * ZROWS, ZROWS)])

        return 0

    lax.fori_loop(0, (ZCH + NS - 1) // NS, wcopy, 0)


def _sc_edge(x, src, dst, e):
    mesh = plsc.VectorSubcoreMesh(core_axis_name="c", subcore_axis_name="s",
                                  num_cores=NC, num_subcores=NS)
    f = pl.kernel(
        _sc_edge_kernel,
        mesh=mesh,
        out_type=jax.ShapeDtypeStruct((NC * N, D), jnp.float32),
        scratch_types=(
            [pltpu.VMEM((C,), jnp.int32)] * 5
            + [pltpu.VMEM((C, D), jnp.float32)] * 4
            + [pltpu.VMEM_SHARED((N, D), jnp.float32)]
            + [pltpu.SemaphoreType.DMA] * 11
        ),
    )
    return f(x, src, dst, e)


def _tc_edge_body(ea_ref, we_ref, be_ref, out_ref):
    out_ref[...] = (
        jnp.dot(ea_ref[...], we_ref[...], preferred_element_type=jnp.float32)
        + be_ref[...]
    )


def _tc_edge(edge_attr, We, be):
    BE = 4000
    return pl.pallas_call(
        _tc_edge_body,
        grid=(E // BE,),
        in_specs=[
            pl.BlockSpec((BE, ED), lambda i: (i, 0)),
            pl.BlockSpec((ED, D), lambda i: (0, 0)),
            pl.BlockSpec((1, D), lambda i: (0, 0)),
        ],
        out_specs=pl.BlockSpec((BE, D), lambda i: (i, 0)),
        out_shape=jax.ShapeDtypeStruct((E, D), jnp.float32),
    )(edge_attr, We, be.reshape(1, D))


def _tc_node_body(x_ref, agg_ref, wa_ref, ba_ref, g_ref, beta_ref,
                  wb_ref, bb_ref, out_ref):
    x = x_ref[...]
    h = x + agg_ref[0] + agg_ref[1]
    h1 = jnp.dot(h, wa_ref[...], preferred_element_type=jnp.float32) + ba_ref[...]
    mean = jnp.mean(h1, axis=0, keepdims=True)
    var = jnp.mean((h1 - mean) ** 2, axis=0, keepdims=True)
    h1 = (h1 - mean) / jnp.sqrt(var + 1e-5) * g_ref[...] + beta_ref[...]
    h1 = jnp.maximum(h1, 0.0)
    out = jnp.dot(h1, wb_ref[...], preferred_element_type=jnp.float32) + bb_ref[...]
    out_ref[...] = jnp.maximum(out + x, 0.0)


def _tc_node(x, agg, Wa, ba, g, beta, Wb, bb):
    return pl.pallas_call(
        _tc_node_body,
        out_shape=jax.ShapeDtypeStruct((N, D), jnp.float32),
    )(x, agg.reshape(NC, N, D), Wa, ba.reshape(1, D), g.reshape(1, D),
      beta.reshape(1, D), Wb, bb.reshape(1, D))


def kernel(x, edge_index, edge_attr, We1, be1, W1a, b1a, g1, beta1, W1b, b1b,
           We2, be2, W2a, b2a, g2, beta2, W2b, b2b):
    src = edge_index[0]
    dst = edge_index[1]
    e1 = _tc_edge(edge_attr, We1, be1)
    e2 = _tc_edge(edge_attr, We2, be2)
    agg1 = _sc_edge(x, src, dst, e1)
    h = _tc_node(x, agg1, W1a, b1a, g1, beta1, W1b, b1b)
    agg2 = _sc_edge(h, src, dst, e2)
    return _tc_node(h, agg2, W2a, b2a, g2, beta2, W2b, b2b)


# speedup vs baseline: 4.8022x; 1.0304x over previous
import functools

import jax
import jax.numpy as jnp
from jax import lax
from jax.experimental import pallas as pl
from jax.experimental.pallas import tpu as pltpu
from jax.experimental.pallas import tpu_sc as plsc

N = 10000
E = 320000
D = 128
ED = 16

NC = 2
NS = 16
NW = NC * NS
EPT = E // NW
C = 80
NCHUNK = EPT // C
ZROWS = 80
ZCH = N // ZROWS
LANES = 16


def _sc_edge_kernel(x_hbm, src_hbm, dst_hbm, e_hbm, out_hbm,
                    sidx0, sidx1, didx0, didx1, didx2, xr0, xr1, er0, er1,
                    acc, gsem0, gsem1, esem0, esem1, lsem0, lsem1,
                    dsem0, dsem1, dsem2, ssem0, ssem1):
    cid = lax.axis_index("c")
    sid = lax.axis_index("s")
    wid = cid * NS + sid

    sidx = (sidx0, sidx1)
    didx = (didx0, didx1, didx2)
    xr = (xr0, xr1)
    er = (er0, er1)
    gsem = (gsem0, gsem1)
    esem = (esem0, esem1)
    lsem = (lsem0, lsem1)
    dsem = (dsem0, dsem1, dsem2)
    ssem = (ssem0, ssem1)

    zero = jnp.zeros((LANES,), jnp.float32)
    zbuf = er0

    def zfill(i, _):
        r = i // (D // LANES)
        col = (i % (D // LANES)) * LANES
        zbuf[r, pl.ds(col, LANES)] = zero
        return 0

    lax.fori_loop(0, ZROWS * (D // LANES), zfill, 0)

    def zcopy(t, _):
        j = sid + t * NS

        @pl.when(j < ZCH)
        def _():
            pltpu.sync_copy(zbuf, acc.at[pl.ds(j * ZROWS, ZROWS)])

        return 0

    lax.fori_loop(0, (ZCH + NS - 1) // NS, zcopy, 0)
    plsc.subcore_barrier()

    base = wid * EPT

    def issue_idx(k, s2, s3):
        pltpu.async_copy(src_hbm.at[pl.ds(base + k * C, C)], sidx[s2],
                         lsem[s2])
        pltpu.async_copy(dst_hbm.at[pl.ds(base + k * C, C)], didx[s3],
                         dsem[s3])

    def wait_idx(k, s2, s3):
        pltpu.make_async_copy(src_hbm.at[pl.ds(base + k * C, C)],
                              sidx[s2], lsem[s2]).wait()
        pltpu.make_async_copy(dst_hbm.at[pl.ds(base + k * C, C)],
                              didx[s3], dsem[s3]).wait()

    def issue_rows(k, slot, s2):
        pltpu.async_copy(x_hbm.at[sidx[s2]], xr[slot], gsem[slot])
        pltpu.async_copy(e_hbm.at[pl.ds(base + k * C, C)], er[slot],
                         esem[slot])

    def wait_rows(k, slot, s2):
        pltpu.make_async_copy(x_hbm.at[sidx[s2]], xr[slot],
                              gsem[slot]).wait()
        pltpu.make_async_copy(e_hbm.at[pl.ds(base + k * C, C)], er[slot],
                              esem[slot]).wait()

    def wait_scatter(slot, s3):
        pltpu.make_async_copy(er[slot], acc.at[didx[s3]],
                              ssem[slot]).wait()

    def compute(slot):
        a = xr[slot]
        b = er[slot]

        def row(i2, _):
            for r in range(2):
                for j in range(D // LANES):
                    s = pl.ds(j * LANES, LANES)
                    i = i2 * 2 + r
                    b[i, s] = jnp.maximum(a[i, s] + b[i, s], 0.0)
            return 0

        lax.fori_loop(0, C // 2, row, 0)

    issue_idx(0, 0, 0)
    wait_idx(0, 0, 0)
    issue_rows(0, 0, 0)
    issue_idx(1, 1, 1)

    def chunk6(k6, _):
        for p in range(6):
            k = k6 * 6 + p
            s2 = p % 2
            o2 = (p + 1) % 2
            s3 = p % 3

            @pl.when(k < NCHUNK)
            def _():
                @pl.when(k + 1 < NCHUNK)
                def _():
                    wait_idx(k + 1, o2, (p + 1) % 3)

                    @pl.when(k >= 1)
                    def _():
                        wait_scatter(o2, (p + 2) % 3)

                    issue_rows(k + 1, o2, o2)

                wait_rows(k, s2, s2)
                compute(s2)
                pltpu.async_copy(er[s2], acc.at[didx[s3]], ssem[s2],
                                 add=True)

                @pl.when(k + 2 < NCHUNK)
                def _():
                    issue_idx(k + 2, s2, (p + 2) % 3)

        return 0

    lax.fori_loop(0, (NCHUNK + 5) // 6, chunk6, 0)

    wait_scatter((NCHUNK - 2) % 2, (NCHUNK - 2) % 3)
    wait_scatter((NCHUNK - 1) % 2, (NCHUNK - 1) % 3)
    plsc.subcore_barrier()

    def wcopy(t, _):
        j = sid + t * NS

        @pl.when(j < ZCH)
        def _():
            pltpu.sync_copy(acc.at[pl.ds(j * ZROWS, ZROWS)],
                            out_hbm.at[pl.ds(cid * N + j * ZROWS, ZROWS)])

        return 0

    lax.fori_loop(0, (ZCH + NS - 1) // NS, wcopy, 0)


def _sc_edge(x, src, dst, e):
    mesh = plsc.VectorSubcoreMesh(core_axis_name="c", subcore_axis_name="s",
                                  num_cores=NC, num_subcores=NS)
    f = pl.kernel(
        _sc_edge_kernel,
        mesh=mesh,
        out_type=jax.ShapeDtypeStruct((NC * N, D), jnp.float32),
        scratch_types=(
            [pltpu.VMEM((C,), jnp.int32)] * 5
            + [pltpu.VMEM((C, D), jnp.float32)] * 4
            + [pltpu.VMEM_SHARED((N, D), jnp.float32)]
            + [pltpu.SemaphoreType.DMA] * 11
        ),
    )
    return f(x, src, dst, e)


def _tc_edge_body(ea_ref, we_ref, be_ref, out_ref):
    out_ref[...] = (
        jnp.dot(ea_ref[...], we_ref[...], preferred_element_type=jnp.float32)
        + be_ref[...]
    )


def _tc_edge(edge_attr, We, be):
    BE = 4000
    return pl.pallas_call(
        _tc_edge_body,
        grid=(E // BE,),
        in_specs=[
            pl.BlockSpec((BE, ED), lambda i: (i, 0)),
            pl.BlockSpec((ED, D), lambda i: (0, 0)),
            pl.BlockSpec((1, D), lambda i: (0, 0)),
        ],
        out_specs=pl.BlockSpec((BE, D), lambda i: (i, 0)),
        out_shape=jax.ShapeDtypeStruct((E, D), jnp.float32),
    )(edge_attr, We, be.reshape(1, D))


def _tc_node_body(x_ref, agg_ref, wa_ref, ba_ref, g_ref, beta_ref,
                  wb_ref, bb_ref, out_ref):
    x = x_ref[...]
    h = x + agg_ref[0] + agg_ref[1]
    h1 = jnp.dot(h, wa_ref[...], preferred_element_type=jnp.float32) + ba_ref[...]
    mean = jnp.mean(h1, axis=0, keepdims=True)
    var = jnp.mean((h1 - mean) ** 2, axis=0, keepdims=True)
    h1 = (h1 - mean) / jnp.sqrt(var + 1e-5) * g_ref[...] + beta_ref[...]
    h1 = jnp.maximum(h1, 0.0)
    out = jnp.dot(h1, wb_ref[...], preferred_element_type=jnp.float32) + bb_ref[...]
    out_ref[...] = jnp.maximum(out + x, 0.0)


def _tc_node(x, agg, Wa, ba, g, beta, Wb, bb):
    return pl.pallas_call(
        _tc_node_body,
        out_shape=jax.ShapeDtypeStruct((N, D), jnp.float32),
    )(x, agg.reshape(NC, N, D), Wa, ba.reshape(1, D), g.reshape(1, D),
      beta.reshape(1, D), Wb, bb.reshape(1, D))


def kernel(x, edge_index, edge_attr, We1, be1, W1a, b1a, g1, beta1, W1b, b1b,
           We2, be2, W2a, b2a, g2, beta2, W2b, b2b):
    src = edge_index[0]
    dst = edge_index[1]
    e1 = _tc_edge(edge_attr, We1, be1)
    e2 = _tc_edge(edge_attr, We2, be2)
    agg1 = _sc_edge(x, src, dst, e1)
    h = _tc_node(x, agg1, W1a, b1a, g1, beta1, W1b, b1b)
    agg2 = _sc_edge(h, src, dst, e2)
    return _tc_node(h, agg2, W2a, b2a, g2, beta2, W2b, b2b)
